# Initial kernel scaffold; baseline (speedup 1.0000x reference)
import jax
import jax.numpy as jnp
from jax.experimental import pallas as pl


def kernel(x, edge_index, W1, b1, W2, b2, W3, b3, Wc, bc):
    raise NotImplementedError("write your pallas kernel here")


# speedup vs baseline: 2.5421x; 2.5421x over previous
import jax
import jax.numpy as jnp
from jax.experimental import pallas as pl


def _gcn_conv(x, s, d, dinv, W, b):
    g = (x @ W) * dinv[:, None]
    agg = jnp.zeros_like(g).at[d].add(g[s])
    return (agg + g) * dinv[:, None] + b


def _final_proj_kernel(h_ref, wc_ref, bc_ref, out_ref):
    out_ref[...] = h_ref[...] @ wc_ref[...] + bc_ref[...]


def kernel(x, edge_index, W1, b1, W2, b2, W3, b3, Wc, bc):
    n = x.shape[0]
    src = edge_index[0]
    dst = edge_index[1]
    deg = jnp.ones((n,), dtype=x.dtype).at[dst].add(1.0)
    dinv = jax.lax.rsqrt(deg)

    h = jnp.tanh(_gcn_conv(x, src, dst, dinv, W1, b1))
    h = jnp.tanh(_gcn_conv(h, src, dst, dinv, W2, b2))
    h = jnp.tanh(_gcn_conv(h, src, dst, dinv, W3, b3))

    out = pl.pallas_call(
        _final_proj_kernel,
        out_shape=jax.ShapeDtypeStruct((n, Wc.shape[1]), x.dtype),
    )(h, Wc, bc)
    return (out, h)


# speedup vs baseline: 14.4619x; 5.6890x over previous
import functools
import jax
import jax.numpy as jnp
from jax import lax
from jax.experimental import pallas as pl
from jax.experimental.pallas import tpu as pltpu
from jax.experimental.pallas import tpu_sc as plsc

N = 10000
E = 320000
NS = 16
NP = 10240
RP = NP // NS
RP4 = RP * 4
EPW = E // NS
EC = 4000
EC4 = EC * 4
NCHUNK = EPW // EC

_mesh = plsc.VectorSubcoreMesh(core_axis_name="c", subcore_axis_name="s",
                               num_cores=1)


def _mm_body(x_ref, w_ref, o_ref):
    o_ref[...] = jnp.dot(x_ref[...], w_ref[...],
                         preferred_element_type=jnp.float32)


def _rsqrt16(x):
    i = plsc.bitcast(x, jnp.int32)
    y = plsc.bitcast(jnp.int32(0x5F3759DF) - (i >> 1), jnp.float32)
    for _ in range(3):
        y = y * (1.5 - 0.5 * x * y * y)
    return y


def _tanh16(x):
    e = jnp.exp(2.0 * x)
    return 1.0 - 2.0 / (e + 1.0)


@functools.partial(
    pl.kernel,
    out_type=[
        jax.ShapeDtypeStruct((NP * 4,), jnp.float32),
        jax.ShapeDtypeStruct((NP * 2,), jnp.float32),
    ],
    mesh=_mesh,
    compiler_params=pltpu.CompilerParams(needs_layout_passes=False),
    scratch_types=[
        pltpu.VMEM_SHARED((NP * 4,), jnp.float32),
        pltpu.VMEM_SHARED((NP * 4,), jnp.float32),
        pltpu.VMEM_SHARED((NP,), jnp.float32),
        pltpu.VMEM((EC4,), jnp.int32),
        pltpu.VMEM((EC4,), jnp.int32),
        pltpu.VMEM((EC,), jnp.int32),
        pltpu.VMEM((EC4,), jnp.float32),
        pltpu.VMEM((EC,), jnp.float32),
        pltpu.VMEM((RP4,), jnp.float32),
        pltpu.VMEM((RP4,), jnp.float32),
        pltpu.VMEM((RP4,), jnp.float32),
        pltpu.VMEM((RP4,), jnp.float32),
        pltpu.VMEM((RP,), jnp.float32),
        pltpu.VMEM((64,), jnp.float32),
        pltpu.SemaphoreType.DMA,
    ],
)
def _gcn_sc(dst_hbm, src4_hbm, dst4_hbm, h1_hbm, par_hbm,
            out_hbm, hout_hbm,
            g_sp, acc_sp, deg_sp, gi_v, si_v, di_v, msg_v, one_v, z_v,
            g_v, a_v, h_v, d_v, p_v, sem):
    wid = lax.axis_index("s")
    nsl = pl.ds(wid * RP4, RP4)
    lanes = lax.iota(jnp.int32, 16)

    def _fill(i, _):
        one_v[pl.ds(i * 16, 16)] = jnp.full((16,), 1.0, jnp.float32)
        return _
    lax.fori_loop(0, EC // 16, _fill, None)

    def _zfill(i, _):
        z_v[pl.ds(i * 16, 16)] = jnp.zeros((16,), jnp.float32)
        return _
    lax.fori_loop(0, RP4 // 16, _zfill, None)

    pltpu.sync_copy(par_hbm, p_v)
    pltpu.sync_copy(z_v.at[pl.ds(0, RP)], deg_sp.at[pl.ds(wid * RP, RP)])
    pltpu.sync_copy(z_v, acc_sp.at[nsl])
    plsc.subcore_barrier()

    for c in range(NCHUNK):
        pltpu.sync_copy(dst_hbm.at[pl.ds(wid * EPW + c * EC, EC)], di_v)
        pltpu.sync_copy(one_v, deg_sp.at[di_v], add=True)
    plsc.subcore_barrier()

    pltpu.sync_copy(deg_sp.at[pl.ds(wid * RP, RP)], d_v)

    def _dinv(i, _):
        s = pl.ds(i * 16, 16)
        d_v[s] = _rsqrt16(d_v[s] + 1.0)
        return _
    lax.fori_loop(0, RP // 16, _dinv, None)

    pltpu.sync_copy(h1_hbm.at[nsl], g_v)

    def _scale(i, _):
        s = pl.ds(i * 16, 16)
        dv = plsc.load_gather(d_v, [(lanes + i * 16) >> 2])
        g_v[s] = g_v[s] * dv
        return _
    lax.fori_loop(0, RP4 // 16, _scale, None)

    pltpu.sync_copy(g_v, g_sp.at[nsl])
    plsc.subcore_barrier()

    for layer in range(3):
        for c in range(NCHUNK):
            esl = pl.ds((wid * EPW + c * EC) * 4, EC4)
            pltpu.sync_copy(src4_hbm.at[esl], gi_v)
            pltpu.async_copy(g_sp.at[gi_v], msg_v, sem).wait()
            pltpu.sync_copy(dst4_hbm.at[esl], si_v)
            pltpu.sync_copy(msg_v, acc_sp.at[si_v], add=True)
        plsc.subcore_barrier()

        pltpu.sync_copy(acc_sp.at[nsl], a_v)
        boff = 48 + 4 * layer

        def _node(i, _):
            s = pl.ds(i * 16, 16)
            flat = lanes + i * 16
            dv = plsc.load_gather(d_v, [flat >> 2])
            bv = plsc.load_gather(p_v, [boff + (flat & 3)])
            h_v[s] = _tanh16(dv * (a_v[s] + g_v[s]) + bv)
            return _
        lax.fori_loop(0, RP4 // 16, _node, None)

        if layer < 2:
            woff = 16 * layer

            def _mm(i, _):
                s = pl.ds(i * 16, 16)
                flat = lanes + i * 16
                nd4 = (flat >> 2) << 2
                cc = flat & 3
                acc = jnp.zeros((16,), jnp.float32)
                for k in range(4):
                    hk = plsc.load_gather(h_v, [nd4 + k])
                    wk = plsc.load_gather(p_v, [woff + 4 * k + cc])
                    acc = acc + hk * wk
                dv = plsc.load_gather(d_v, [flat >> 2])
                g_v[s] = acc * dv
                return _
            lax.fori_loop(0, RP4 // 16, _mm, None)

            pltpu.sync_copy(g_v, g_sp.at[nsl])
            pltpu.sync_copy(z_v, acc_sp.at[nsl])
        else:
            def _cls(i, _):
                s = pl.ds(i * 16, 16)
                flat = lanes + i * 16
                nd4 = (flat >> 2) << 2
                cc = flat & 3
                acc = plsc.load_gather(p_v, [60 + cc])
                for k in range(4):
                    hk = plsc.load_gather(h_v, [nd4 + k])
                    wk = plsc.load_gather(p_v, [32 + 4 * k + cc])
                    acc = acc + hk * wk
                a_v[s] = acc
                return _
            lax.fori_loop(0, RP4 // 16, _cls, None)

            pltpu.sync_copy(a_v, out_hbm.at[nsl])

            def _hout(i, _):
                s = pl.ds(i * 16, 16)
                f2 = lanes + i * 16
                g_v[s] = plsc.load_gather(h_v, [((f2 >> 1) << 2) + (f2 & 1)])
                return _
            lax.fori_loop(0, RP * 2 // 16, _hout, None)

            pltpu.sync_copy(g_v.at[pl.ds(0, RP * 2)],
                            hout_hbm.at[pl.ds(wid * RP * 2, RP * 2)])
        plsc.subcore_barrier()


def kernel(x, edge_index, W1, b1, W2, b2, W3, b3, Wc, bc):
    src = edge_index[0]
    dst = edge_index[1]

    h1 = pl.pallas_call(
        _mm_body,
        out_shape=jax.ShapeDtypeStruct((N, 4), jnp.float32),
    )(x, W1)

    h1f = jnp.pad(h1, ((0, NP - N), (0, 0))).ravel()
    four = jnp.arange(4, dtype=jnp.int32)
    src4 = (4 * src[:, None] + four).ravel()
    dst4 = (4 * dst[:, None] + four).ravel()
    w3p = jnp.pad(W3, ((0, 0), (0, 2)))
    wcp = jnp.pad(Wc, ((0, 2), (0, 0)))
    b3p = jnp.pad(b3, (0, 2))
    par = jnp.concatenate([W2.ravel(), w3p.ravel(), wcp.ravel(),
                           b1, b2, b3p, bc]).astype(jnp.float32)

    out_f, h_f = _gcn_sc(dst, src4, dst4, h1f, par)
    out = out_f.reshape(NP, 4)[:N]
    h = h_f.reshape(NP, 2)[:N]
    return (out, h)
